# baseline (speedup 1.0000x reference)
import functools

import jax
import jax.numpy as jnp
from jax import lax
from jax.experimental import pallas as pl
from jax.experimental.pallas import tpu as pltpu
from jax.experimental.pallas import tpu_sc as plsc

CLASSES = 100000
DIMS = 64
BATCH = 16384
EPS = 1e-09

NC, NS, L = 2, 16, 16
NW = NC * NS
BPW = BATCH // NW
HPW = BPW // 2
RSTR = 65
PACK_R = 512
HCLS = 98 * PACK_R


def _pack_body(a_ref, b_ref, out_ref):
    eye = jnp.eye(DIMS, dtype=jnp.float32)
    dn = (((0,), (0,)), ((), ()))
    out_ref[:, 0:DIMS] = jax.lax.dot_general(
        a_ref[...], eye, dn, preferred_element_type=jnp.float32)
    out_ref[:, DIMS:2 * DIMS] = jax.lax.dot_general(
        b_ref[...], eye, dn, preferred_element_type=jnp.float32)


_pack_table = pl.pallas_call(
    _pack_body,
    grid=(HCLS // PACK_R,),
    in_specs=[
        pl.BlockSpec((DIMS, PACK_R), lambda i: (0, i)),
        pl.BlockSpec((DIMS, PACK_R), lambda i: (0, i + HCLS // PACK_R)),
    ],
    out_specs=pl.BlockSpec((PACK_R, 2 * DIMS), lambda i: (i, 0)),
    out_shape=jax.ShapeDtypeStruct((HCLS, 2 * DIMS), jnp.float32),
)


def _loss_body(pred_hbm, yt_hbm, pol_hbm, out_hbm,
               rows_v, pred_v, idx_v, stage_v, rsem, psem):
    wid = lax.axis_index("s") * NC + lax.axis_index("c")
    base = wid * BPW
    lane = lax.iota(jnp.int32, L)

    pltpu.sync_copy(yt_hbm.at[pl.ds(wid * 8, 8)], idx_v)

    def rows_fire(g, c):
        civ = plsc.bitcast(idx_v[g >> 2, pl.ds((g & 3) * L, L)], jnp.int32)
        s0 = g * L
        for l in range(L):
            ci = civ[l]
            hi = (ci >= HCLS).astype(jnp.int32)
            src = pol_hbm.at[ci - hi * HCLS, pl.ds(hi * DIMS, DIMS)]
            pltpu.make_async_copy(
                src, rows_v.at[s0 + l, pl.ds(0, DIMS)], rsem).start()
        return c

    def pred_fire(phase):
        def fire(d, c):
            pltpu.make_async_copy(
                pred_hbm.at[d, pl.ds(base + phase * HPW, HPW)],
                pred_v.at[d], psem).start()
            return c
        return fire

    def rows_drain(i, c):
        pltpu.make_async_copy(pol_hbm.at[0, pl.ds(0, DIMS)],
                              rows_v.at[0, pl.ds(0, DIMS)], rsem).wait()
        return c

    def pred_drain(d, c):
        pltpu.make_async_copy(pred_hbm.at[0, pl.ds(0, HPW)],
                              pred_v.at[0], psem).wait()
        return c

    lax.fori_loop(0, BPW // L, rows_fire, jnp.int32(0))
    lax.fori_loop(0, DIMS, pred_fire(0), jnp.int32(0))
    lax.fori_loop(0, BPW, rows_drain, jnp.int32(0))
    lax.fori_loop(0, DIMS, pred_drain, jnp.int32(0))

    half = jnp.float32(0.5)
    three_half = jnp.float32(1.5)
    one = jnp.float32(1.0)

    def make_group_body(goff):
        def group_body(g, acc):
            s = lane + g * L
            col = (g - goff) * L
            dot = [None] * 4
            n1 = [None] * 4
            n2 = [None] * 4
            for d in range(DIMS):
                cd = jnp.full((L,), d, jnp.int32)
                pv = pred_v[d, pl.ds(col, L)]
                tv = plsc.load_gather(rows_v, [s, cd])
                k = d & 3
                if dot[k] is None:
                    dot[k], n1[k], n2[k] = pv * tv, pv * pv, tv * tv
                else:
                    dot[k] = dot[k] + pv * tv
                    n1[k] = n1[k] + pv * pv
                    n2[k] = n2[k] + tv * tv
            dotv = (dot[0] + dot[1]) + (dot[2] + dot[3])
            n1v = (n1[0] + n1[1]) + (n1[2] + n1[3])
            n2v = (n2[0] + n2[1]) + (n2[2] + n2[3])
            prod = jnp.maximum(n1v * n2v, jnp.float32(1e-30))
            bits = plsc.bitcast(prod, jnp.int32)
            y = plsc.bitcast(jnp.int32(0x5F3759DF) - (bits >> 1),
                             jnp.float32)
            for _ in range(3):
                y = y * (three_half - half * prod * y * y)
            scale = jnp.where(prod >= jnp.float32(EPS * EPS), y,
                              jnp.float32(1.0 / EPS))
            cos = dotv * scale
            e = one - cos
            return acc + e * e
        return group_body

    acc = lax.fori_loop(0, HPW // L, make_group_body(0),
                        jnp.zeros((L,), jnp.float32))
    lax.fori_loop(0, DIMS, pred_fire(1), jnp.int32(0))
    lax.fori_loop(0, DIMS, pred_drain, jnp.int32(0))
    acc = lax.fori_loop(HPW // L, BPW // L, make_group_body(HPW // L), acc)

    stage_v[...] = acc * jnp.float32(1.0 / BATCH)
    pltpu.sync_copy(stage_v, out_hbm.at[wid])


_sc_loss = functools.partial(
    pl.kernel,
    mesh=plsc.VectorSubcoreMesh(core_axis_name="c", subcore_axis_name="s"),
    out_type=jax.ShapeDtypeStruct((NW, L), jnp.float32),
    compiler_params=pltpu.CompilerParams(needs_layout_passes=False),
    scratch_types=[
        pltpu.VMEM((BPW, RSTR), jnp.float32),
        pltpu.VMEM((DIMS, HPW), jnp.float32),
        pltpu.VMEM((8, DIMS), jnp.float32),
        pltpu.VMEM((L,), jnp.float32),
        pltpu.SemaphoreType.DMA,
        pltpu.SemaphoreType.DMA,
    ],
)(_loss_body)


def kernel(y_pred, y_true, polars):
    yt = lax.bitcast_convert_type(y_true.astype(jnp.int32),
                                  jnp.float32).reshape(BATCH // DIMS, DIMS)
    pol_t = polars.T
    pred_t = y_pred.T
    packed = _pack_table(pol_t, pol_t)
    partials = _sc_loss(pred_t, yt, packed)
    return jnp.sum(partials)


# speedup vs baseline: 1.0633x; 1.0633x over previous
import functools

import jax
import jax.numpy as jnp
from jax import lax
from jax.experimental import pallas as pl
from jax.experimental.pallas import tpu as pltpu
from jax.experimental.pallas import tpu_sc as plsc

CLASSES = 100000
DIMS = 64
BATCH = 16384
EPS = 1e-09

NC, NS, L = 2, 16, 16
NW = NC * NS
BPW = BATCH // NW
HPW = BPW // 2
HCLS = 390 * 128
NPACK = 392 * 128
CHUNK = 128
NBLK = 13


def _pack_body(pol_hbm, out_hbm, in_lo, in_hi, out_v, sem):
    wid = lax.axis_index("s") * NC + lax.axis_index("c")
    lane = lax.iota(jnp.int32, L)

    def chunk_body(k, c):
        row0 = jnp.minimum(wid * NBLK + k, NPACK // CHUNK - 1) * CHUNK
        for d8 in range(DIMS // 8):
            pltpu.make_async_copy(
                pol_hbm.at[pl.ds(d8 * 8, 8), pl.ds(row0, CHUNK)],
                in_lo.at[pl.ds(d8 * 8, 8)], sem).start()
            pltpu.make_async_copy(
                pol_hbm.at[pl.ds(d8 * 8, 8), pl.ds(row0 + HCLS, CHUNK)],
                in_hi.at[pl.ds(d8 * 8, 8)], sem).start()
        for d8 in range(2 * DIMS // 8):
            pltpu.make_async_copy(pol_hbm.at[pl.ds(0, 8), pl.ds(0, CHUNK)],
                                  in_lo.at[pl.ds(0, 8)], sem).wait()

        def group_body(g, cc):
            clv = lane + g * L
            for d in range(DIMS):
                rot = (lane + d) & (DIMS - 1)
                plsc.store_scatter(out_v, [clv, rot],
                                   plsc.load_gather(in_lo, [rot, clv]))
                plsc.store_scatter(out_v, [clv, rot + DIMS],
                                   plsc.load_gather(in_hi, [rot, clv]))
            return cc

        lax.fori_loop(0, CHUNK // L, group_body, jnp.int32(0))
        pltpu.sync_copy(out_v, out_hbm.at[pl.ds(row0, CHUNK)])
        return c

    lax.fori_loop(0, NBLK, chunk_body, jnp.int32(0))


_sc_pack = functools.partial(
    pl.kernel,
    mesh=plsc.VectorSubcoreMesh(core_axis_name="c", subcore_axis_name="s"),
    out_type=jax.ShapeDtypeStruct((NPACK, 2 * DIMS), jnp.float32),
    compiler_params=pltpu.CompilerParams(
        needs_layout_passes=False, disable_bounds_checks=True),
    scratch_types=[
        pltpu.VMEM((DIMS, CHUNK), jnp.float32),
        pltpu.VMEM((DIMS, CHUNK), jnp.float32),
        pltpu.VMEM((CHUNK, 2 * DIMS), jnp.float32),
        pltpu.SemaphoreType.DMA,
    ],
)(_pack_body)


def _loss_body(pred_hbm, yt_hbm, pol_hbm, out_hbm,
               rows_v, pred_v, stage_v, rsem, psem):
    wid = lax.axis_index("s") * NC + lax.axis_index("c")
    base = wid * BPW
    lane = lax.iota(jnp.int32, L)

    pltpu.sync_copy(yt_hbm.at[pl.ds(wid * 8, 8)], pred_v.at[pl.ds(0, 8)])

    def rows_fire(g, c):
        civ = plsc.bitcast(pred_v[g >> 2, pl.ds((g & 3) * L, L)], jnp.int32)
        s0 = g * L
        for l in range(L):
            ci = civ[l]
            hi = (ci >= HCLS).astype(jnp.int32)
            src = pol_hbm.at[ci - hi * HCLS, pl.ds(hi * DIMS, DIMS)]
            pltpu.make_async_copy(src, rows_v.at[s0 + l], rsem).start()
        return c

    def pred_fire(i, c):
        pltpu.make_async_copy(pred_hbm.at[base + i],
                              pred_v.at[i & (HPW - 1)], psem).start()
        return c

    def rows_drain(i, c):
        pltpu.make_async_copy(pol_hbm.at[0, pl.ds(0, DIMS)],
                              rows_v.at[i], rsem).wait()
        return c

    def pred_drain(i, c):
        pltpu.make_async_copy(pred_hbm.at[0], pred_v.at[0], psem).wait()
        return c

    lax.fori_loop(0, BPW // L, rows_fire, jnp.int32(0))
    lax.fori_loop(0, HPW, pred_fire, jnp.int32(0))
    lax.fori_loop(0, BPW, rows_drain, jnp.int32(0))
    lax.fori_loop(0, HPW, pred_drain, jnp.int32(0))

    half = jnp.float32(0.5)
    three_half = jnp.float32(1.5)
    one = jnp.float32(1.0)

    def make_group_body(pred_base):
        def group_body(g, acc):
            s = lane + g * L
            sp = s - pred_base
            dot = [None] * 4
            n1 = [None] * 4
            n2 = [None] * 4
            for t in range(DIMS):
                ct = (lane + t) & (DIMS - 1)
                pv = plsc.load_gather(pred_v, [sp, ct])
                tv = plsc.load_gather(rows_v, [s, ct])
                k = t & 3
                if dot[k] is None:
                    dot[k], n1[k], n2[k] = pv * tv, pv * pv, tv * tv
                else:
                    dot[k] = dot[k] + pv * tv
                    n1[k] = n1[k] + pv * pv
                    n2[k] = n2[k] + tv * tv
            dotv = (dot[0] + dot[1]) + (dot[2] + dot[3])
            n1v = (n1[0] + n1[1]) + (n1[2] + n1[3])
            n2v = (n2[0] + n2[1]) + (n2[2] + n2[3])
            prod = jnp.maximum(n1v * n2v, jnp.float32(1e-30))
            bits = plsc.bitcast(prod, jnp.int32)
            y = plsc.bitcast(jnp.int32(0x5F3759DF) - (bits >> 1),
                             jnp.float32)
            for _ in range(3):
                y = y * (three_half - half * prod * y * y)
            scale = jnp.where(prod >= jnp.float32(EPS * EPS), y,
                              jnp.float32(1.0 / EPS))
            cos = dotv * scale
            e = one - cos
            return acc + e * e
        return group_body

    acc = lax.fori_loop(0, HPW // L, make_group_body(0),
                        jnp.zeros((L,), jnp.float32))
    lax.fori_loop(HPW, BPW, pred_fire, jnp.int32(0))
    lax.fori_loop(0, HPW, pred_drain, jnp.int32(0))
    acc = lax.fori_loop(HPW // L, BPW // L, make_group_body(HPW), acc)

    stage_v[...] = acc * jnp.float32(1.0 / BATCH)
    pltpu.sync_copy(stage_v, out_hbm.at[wid])


_sc_loss = functools.partial(
    pl.kernel,
    mesh=plsc.VectorSubcoreMesh(core_axis_name="c", subcore_axis_name="s"),
    out_type=jax.ShapeDtypeStruct((NW, L), jnp.float32),
    compiler_params=pltpu.CompilerParams(needs_layout_passes=False),
    scratch_types=[
        pltpu.VMEM((BPW, DIMS), jnp.float32),
        pltpu.VMEM((HPW, DIMS), jnp.float32),
        pltpu.VMEM((L,), jnp.float32),
        pltpu.SemaphoreType.DMA,
        pltpu.SemaphoreType.DMA,
    ],
)(_loss_body)


def kernel(y_pred, y_true, polars):
    yt = lax.bitcast_convert_type(y_true.astype(jnp.int32),
                                  jnp.float32).reshape(BATCH // DIMS, DIMS)
    packed = _sc_pack(polars.T)
    partials = _sc_loss(y_pred, yt, packed)
    return jnp.sum(partials)


# speedup vs baseline: 1.6227x; 1.5261x over previous
import functools

import jax
import jax.numpy as jnp
from jax import lax
from jax.experimental import pallas as pl
from jax.experimental.pallas import tpu as pltpu
from jax.experimental.pallas import tpu_sc as plsc

CLASSES = 100000
DIMS = 64
BATCH = 16384
EPS = 1e-09

NC, NS, L = 2, 16, 16
NW = NC * NS
BPW = BATCH // NW
HPW = BPW // 2


def _loss_body(pred_hbm, yt_hbm, pol_hbm, out_hbm,
               rows_v, pred_v, stage_v, rsem, psem):
    wid = lax.axis_index("s") * NC + lax.axis_index("c")
    base = wid * BPW
    lane = lax.iota(jnp.int32, L)

    pltpu.sync_copy(yt_hbm.at[pl.ds(wid * 8, 8)], pred_v.at[pl.ds(0, 8)])

    def rows_fire(g, c):
        civ = plsc.bitcast(pred_v[g >> 2, pl.ds((g & 3) * L, L)], jnp.int32)
        s0 = g * L
        for l in range(L):
            pltpu.make_async_copy(pol_hbm.at[civ[l]], rows_v.at[s0 + l],
                                  rsem).start()
        return c

    def pred_fire(i, c):
        pltpu.make_async_copy(pred_hbm.at[base + i],
                              pred_v.at[i & (HPW - 1)], psem).start()
        return c

    def rows_drain(i, c):
        pltpu.make_async_copy(pol_hbm.at[0], rows_v.at[i], rsem).wait()
        return c

    def pred_drain(i, c):
        pltpu.make_async_copy(pred_hbm.at[0], pred_v.at[0], psem).wait()
        return c

    lax.fori_loop(0, BPW // L, rows_fire, jnp.int32(0))
    lax.fori_loop(0, HPW, pred_fire, jnp.int32(0))
    lax.fori_loop(0, BPW, rows_drain, jnp.int32(0))
    lax.fori_loop(0, HPW, pred_drain, jnp.int32(0))

    half = jnp.float32(0.5)
    three_half = jnp.float32(1.5)
    one = jnp.float32(1.0)

    def make_group_body(pred_base):
        def group_body(g, acc):
            s = lane + g * L
            sp = s - pred_base
            dot = [None] * 4
            n1 = [None] * 4
            n2 = [None] * 4
            for t in range(DIMS):
                ct = (lane + t) & (DIMS - 1)
                pv = plsc.load_gather(pred_v, [sp, ct])
                tv = plsc.load_gather(rows_v, [s, ct])
                k = t & 3
                if dot[k] is None:
                    dot[k], n1[k], n2[k] = pv * tv, pv * pv, tv * tv
                else:
                    dot[k] = dot[k] + pv * tv
                    n1[k] = n1[k] + pv * pv
                    n2[k] = n2[k] + tv * tv
            dotv = (dot[0] + dot[1]) + (dot[2] + dot[3])
            n1v = (n1[0] + n1[1]) + (n1[2] + n1[3])
            n2v = (n2[0] + n2[1]) + (n2[2] + n2[3])
            prod = jnp.maximum(n1v * n2v, jnp.float32(1e-30))
            bits = plsc.bitcast(prod, jnp.int32)
            y = plsc.bitcast(jnp.int32(0x5F3759DF) - (bits >> 1),
                             jnp.float32)
            for _ in range(3):
                y = y * (three_half - half * prod * y * y)
            scale = jnp.where(prod >= jnp.float32(EPS * EPS), y,
                              jnp.float32(1.0 / EPS))
            cos = dotv * scale
            e = one - cos
            return acc + e * e
        return group_body

    acc = lax.fori_loop(0, HPW // L, make_group_body(0),
                        jnp.zeros((L,), jnp.float32))
    lax.fori_loop(HPW, BPW, pred_fire, jnp.int32(0))
    lax.fori_loop(0, HPW, pred_drain, jnp.int32(0))
    acc = lax.fori_loop(HPW // L, BPW // L, make_group_body(HPW), acc)

    stage_v[...] = acc * jnp.float32(1.0 / BATCH)
    pltpu.sync_copy(stage_v, out_hbm.at[wid])


_sc_loss = functools.partial(
    pl.kernel,
    mesh=plsc.VectorSubcoreMesh(core_axis_name="c", subcore_axis_name="s"),
    out_type=jax.ShapeDtypeStruct((NW, L), jnp.float32),
    compiler_params=pltpu.CompilerParams(needs_layout_passes=False),
    scratch_types=[
        pltpu.VMEM((BPW, DIMS), jnp.float32),
        pltpu.VMEM((HPW, DIMS), jnp.float32),
        pltpu.VMEM((L,), jnp.float32),
        pltpu.SemaphoreType.DMA,
        pltpu.SemaphoreType.DMA,
    ],
)(_loss_body)


def kernel(y_pred, y_true, polars):
    yt = lax.bitcast_convert_type(y_true.astype(jnp.int32),
                                  jnp.float32).reshape(BATCH // DIMS, DIMS)
    partials = _sc_loss(y_pred, yt, polars)
    return jnp.sum(partials)
